# baseline (speedup 1.0000x reference)
import functools

import jax
import jax.numpy as jnp
from jax import lax
from jax.experimental import pallas as pl
from jax.experimental.pallas import tpu as pltpu
from jax.experimental.pallas import tpu_sc as plsc

FEAT_DIM = 64
BATCH = 16384
NCLASS = 1000000
LAMBDA_C = 2.0
LANES = 16
WIN = 512
PITCH = 513
WSTART_MAX = ((NCLASS - WIN) // 128) * 128
TAIL0 = (NCLASS // 128) * 128
TAIL_W = NCLASS - TAIL0


def _make_partials():
    info = plsc.get_sparse_core_info()
    nc, ns = info.num_cores, info.num_subcores
    nw = nc * ns
    b_per_w = BATCH // nw

    mesh = plsc.VectorSubcoreMesh(core_axis_name="c", subcore_axis_name="s")

    @functools.partial(
        pl.kernel,
        mesh=mesh,
        out_type=jax.ShapeDtypeStruct((nw * LANES,), jnp.float32),
        compiler_params=pltpu.CompilerParams(
            use_tc_tiling_on_sc=True, needs_layout_passes=False),
        scratch_types=[
            pltpu.VMEM((b_per_w,), jnp.int32),
            pltpu.VMEM((FEAT_DIM, PITCH), jnp.float32),
            pltpu.VMEM((FEAT_DIM, PITCH), jnp.float32),
            pltpu.VMEM((FEAT_DIM, b_per_w), jnp.float32),
            pltpu.VMEM((FEAT_DIM, TAIL_W), jnp.float32),
            pltpu.VMEM((LANES,), jnp.float32),
            pltpu.SemaphoreType.DMA,
            pltpu.SemaphoreType.DMA,
        ],
    )
    def partials(featT_hbm, slab_hbm, ct_hbm, out_hbm,
                 labv, wbufa, wbufb, fb, tailbuf, acc_v, sema, semb):
        wid = lax.axis_index("s") * nc + lax.axis_index("c")
        base = pl.multiple_of(wid * b_per_w, 128)
        obase = pl.multiple_of(wid * LANES, 8)

        pltpu.sync_copy(slab_hbm.at[pl.ds(base, b_per_w)], labv)
        pltpu.sync_copy(featT_hbm.at[:, pl.ds(base, b_per_w)], fb)
        pltpu.sync_copy(ct_hbm.at[:, pl.ds(TAIL0, TAIL_W)], tailbuf)

        zero = jnp.zeros((LANES,), jnp.float32)
        lanes_i = lax.iota(jnp.int32, LANES)
        maxp = b_per_w - 1

        def process(buf, wlo, hi, ptr, accs):
            a = list(accs)
            cnt = jnp.int32(0)
            for half in range(2):
                p = ptr + half * LANES + lanes_i
                cpos = jnp.minimum(p, maxp)
                lv = plsc.load_gather(labv, [cpos])
                sel = jnp.logical_and(p < b_per_w, lv < hi)
                sf = jnp.where(sel, 1.0, 0.0).astype(jnp.float32)
                off = jnp.clip(lv - wlo, 0, buf.shape[1] - 1)
                for f in range(FEAT_DIM):
                    fsplat = jnp.full((LANES,), f, jnp.int32)
                    cvec = plsc.load_gather(buf, [fsplat, off])
                    fvec = plsc.load_gather(fb, [fsplat, cpos])
                    d = fvec - cvec
                    a[f % 4] = a[f % 4] + sf * (d * d)
                cnt = cnt + jnp.sum(sel.astype(jnp.int32))
            return tuple(a), cnt

        def next_lab(nptr):
            cpos = jnp.minimum(nptr + lanes_i, maxp)
            return jnp.min(plsc.load_gather(labv, [cpos]))

        def winstart(lab):
            cw = jnp.minimum((lab >> 7) << 7, WSTART_MAX)
            return pl.multiple_of(cw, 128)

        def start_fetch(buf, sem, cw):
            pltpu.make_async_copy(ct_hbm.at[:, pl.ds(cw, WIN)],
                                  buf.at[:, pl.ds(0, WIN)], sem).start()

        def wait_fetch(buf, sem):
            pltpu.make_async_copy(ct_hbm.at[:, pl.ds(0, WIN)],
                                  buf.at[:, pl.ds(0, WIN)], sem).wait()

        def phase(buf, sem, st):
            ptr, lab, wx, wo = st[0], st[1], st[2], st[3]
            wait_fetch(buf, sem)
            needed = winstart(lab)
            valid = wx == needed
            hi = jnp.where(valid, wx + WIN, jnp.int32(-1))
            accs, cnt = process(buf, wx, hi, ptr, st[4:])
            nptr = ptr + cnt
            nlab = next_lab(nptr)
            spec = jnp.where(valid,
                             jnp.minimum(wo + WIN, WSTART_MAX),
                             winstart(nlab))
            spec = pl.multiple_of(spec, 128)
            start_fetch(buf, sem, spec)
            return (nptr, nlab, wo, spec) + accs

        def main_cond(st):
            ptr, lab = st[0], st[1]
            return jnp.logical_and(ptr < b_per_w, lab < TAIL0)

        def main_body(st):
            st = phase(wbufa, sema, st)
            st = phase(wbufb, semb, st)
            return st

        def tail_cond(st):
            return st[0] < b_per_w

        def tail_body(st):
            ptr = st[0]
            accs, cnt = process(tailbuf, TAIL0, NCLASS, ptr, st[4:])
            nptr = ptr + cnt
            return (nptr, st[1], st[2], st[3]) + accs

        lab0 = next_lab(jnp.int32(0))
        cw0 = winstart(lab0)
        cw1 = pl.multiple_of(jnp.minimum(cw0 + WIN, WSTART_MAX), 128)
        start_fetch(wbufa, sema, cw0)
        start_fetch(wbufb, semb, cw1)
        st0 = (jnp.int32(0), lab0, cw0, cw1, zero, zero, zero, zero)
        st1 = lax.while_loop(main_cond, main_body, st0)
        wait_fetch(wbufa, sema)
        wait_fetch(wbufb, semb)
        st2 = lax.while_loop(tail_cond, tail_body, st1)

        a0, a1, a2, a3 = st2[4:]
        acc_v[...] = (a0 + a1) + (a2 + a3)
        pltpu.sync_copy(acc_v, out_hbm.at[pl.ds(obase, LANES)])

    return partials, nw


def kernel(feat, label, centers):
    partials, nw = _make_partials()
    iot = lax.iota(jnp.int32, BATCH)
    slab, order = lax.sort_key_val(label, iot)
    feat_s = jnp.take(feat, order, axis=0)
    parts = partials(feat_s.T, slab, centers.T)
    total = jnp.sum(parts)
    return LAMBDA_C / 2.0 / BATCH * jnp.sqrt(total)


# speedup vs baseline: 1.6672x; 1.6672x over previous
import functools

import jax
import jax.numpy as jnp
from jax import lax
from jax.experimental import pallas as pl
from jax.experimental.pallas import tpu as pltpu
from jax.experimental.pallas import tpu_sc as plsc

FEAT_DIM = 64
BATCH = 16384
NCLASS = 1000000
LAMBDA_C = 2.0
LANES = 16
WIN = 512
PITCH = 513
WSTART_MAX = ((NCLASS - WIN) // 128) * 128
TAIL0 = (NCLASS // 128) * 128
TAIL_W = NCLASS - TAIL0


def _make_partials():
    info = plsc.get_sparse_core_info()
    nc, ns = info.num_cores, info.num_subcores
    nw = nc * ns
    b_per_w = BATCH // nw

    mesh = plsc.VectorSubcoreMesh(core_axis_name="c", subcore_axis_name="s")

    @functools.partial(
        pl.kernel,
        mesh=mesh,
        out_type=jax.ShapeDtypeStruct((nw * LANES,), jnp.float32),
        compiler_params=pltpu.CompilerParams(
            use_tc_tiling_on_sc=True, needs_layout_passes=False),
        scratch_types=[
            pltpu.VMEM((b_per_w,), jnp.int32),
            pltpu.VMEM((FEAT_DIM, PITCH), jnp.float32),
            pltpu.VMEM((FEAT_DIM, PITCH), jnp.float32),
            pltpu.VMEM((FEAT_DIM, b_per_w), jnp.float32),
            pltpu.VMEM((FEAT_DIM, TAIL_W), jnp.float32),
            pltpu.VMEM((LANES,), jnp.float32),
            pltpu.SemaphoreType.DMA,
            pltpu.SemaphoreType.DMA,
        ],
    )
    def partials(featT_hbm, slab_hbm, ct_hbm, out_hbm,
                 labv, wbufa, wbufb, fb, tailbuf, acc_v, sema, semb):
        wid = lax.axis_index("s") * nc + lax.axis_index("c")
        base = pl.multiple_of(wid * b_per_w, 128)
        obase = pl.multiple_of(wid * LANES, 8)

        pltpu.sync_copy(slab_hbm.at[pl.ds(base, b_per_w)], labv)
        pltpu.sync_copy(featT_hbm.at[:, pl.ds(base, b_per_w)], fb)
        pltpu.sync_copy(ct_hbm.at[:, pl.ds(TAIL0, TAIL_W)], tailbuf)

        zero = jnp.zeros((LANES,), jnp.float32)
        lanes_i = lax.iota(jnp.int32, LANES)
        maxp = b_per_w - 1

        def process(buf, wlo, hi, ptr, accs):
            a = list(accs)
            cnt = jnp.int32(0)
            for half in range(2):
                p = ptr + half * LANES + lanes_i
                cpos = jnp.minimum(p, maxp)
                lv = plsc.load_gather(labv, [cpos])
                sel = jnp.logical_and(p < b_per_w, lv < hi)
                sf = jnp.where(sel, 1.0, 0.0).astype(jnp.float32)
                off = jnp.clip(lv - wlo, 0, buf.shape[1] - 1)
                for f in range(FEAT_DIM):
                    fsplat = jnp.full((LANES,), f, jnp.int32)
                    cvec = plsc.load_gather(buf, [fsplat, off])
                    fvec = plsc.load_gather(fb, [fsplat, cpos])
                    d = fvec - cvec
                    a[f % 4] = a[f % 4] + sf * (d * d)
                cnt = cnt + jnp.sum(sel.astype(jnp.int32))
            return tuple(a), cnt

        def next_lab(nptr):
            cpos = jnp.minimum(nptr + lanes_i, maxp)
            return jnp.min(plsc.load_gather(labv, [cpos]))

        def winstart(lab):
            cw = jnp.minimum((lab >> 7) << 7, WSTART_MAX)
            return pl.multiple_of(cw, 128)

        def start_fetch(buf, sem, cw):
            pltpu.make_async_copy(ct_hbm.at[:, pl.ds(cw, WIN)],
                                  buf.at[:, pl.ds(0, WIN)], sem).start()

        def wait_fetch(buf, sem):
            pltpu.make_async_copy(ct_hbm.at[:, pl.ds(0, WIN)],
                                  buf.at[:, pl.ds(0, WIN)], sem).wait()

        def phase(buf, sem, st):
            ptr, lab, wx, wo = st[0], st[1], st[2], st[3]
            wait_fetch(buf, sem)
            valid = jnp.logical_and(lab >= wx, lab < wx + WIN)
            hi = jnp.where(valid, wx + WIN, jnp.int32(-1))
            accs, cnt = process(buf, wx, hi, ptr, st[4:])
            nptr = ptr + cnt
            nlab = next_lab(nptr)
            spec = jnp.where(valid,
                             jnp.minimum(wo + WIN, WSTART_MAX),
                             winstart(nlab))
            spec = pl.multiple_of(spec, 128)
            start_fetch(buf, sem, spec)
            return (nptr, nlab, wo, spec) + accs

        def main_cond(st):
            ptr, lab = st[0], st[1]
            return jnp.logical_and(ptr < b_per_w, lab < TAIL0)

        def main_body(st):
            st = phase(wbufa, sema, st)
            st = phase(wbufb, semb, st)
            return st

        def tail_cond(st):
            return st[0] < b_per_w

        def tail_body(st):
            ptr = st[0]
            accs, cnt = process(tailbuf, TAIL0, NCLASS, ptr, st[4:])
            nptr = ptr + cnt
            return (nptr, st[1], st[2], st[3]) + accs

        lab0 = next_lab(jnp.int32(0))
        cw0 = winstart(lab0)
        cw1 = pl.multiple_of(jnp.minimum(cw0 + WIN, WSTART_MAX), 128)
        start_fetch(wbufa, sema, cw0)
        start_fetch(wbufb, semb, cw1)
        st0 = (jnp.int32(0), lab0, cw0, cw1, zero, zero, zero, zero)
        st1 = lax.while_loop(main_cond, main_body, st0)
        wait_fetch(wbufa, sema)
        wait_fetch(wbufb, semb)
        st2 = lax.while_loop(tail_cond, tail_body, st1)

        a0, a1, a2, a3 = st2[4:]
        acc_v[...] = (a0 + a1) + (a2 + a3)
        pltpu.sync_copy(acc_v, out_hbm.at[pl.ds(obase, LANES)])

    return partials, nw


def kernel(feat, label, centers):
    partials, nw = _make_partials()
    iot = lax.iota(jnp.int32, BATCH)
    slab, order = lax.sort_key_val(label, iot)
    feat_s = jnp.take(feat, order, axis=0)
    parts = partials(feat_s.T, slab, centers.T)
    total = jnp.sum(parts)
    return LAMBDA_C / 2.0 / BATCH * jnp.sqrt(total)


# speedup vs baseline: 1.7588x; 1.0549x over previous
import functools

import jax
import jax.numpy as jnp
from jax import lax
from jax.experimental import pallas as pl
from jax.experimental.pallas import tpu as pltpu
from jax.experimental.pallas import tpu_sc as plsc

FEAT_DIM = 64
BATCH = 16384
NCLASS = 1000000
LAMBDA_C = 2.0
LANES = 16
WIN = 1152
PITCH = 1153
WSTART_MAX = ((NCLASS - WIN) // 128) * 128
TAIL0 = (NCLASS // 128) * 128
TAIL_W = NCLASS - TAIL0


def _make_partials():
    info = plsc.get_sparse_core_info()
    nc, ns = info.num_cores, info.num_subcores
    nw = nc * ns
    b_per_w = BATCH // nw

    mesh = plsc.VectorSubcoreMesh(core_axis_name="c", subcore_axis_name="s")

    @functools.partial(
        pl.kernel,
        mesh=mesh,
        out_type=jax.ShapeDtypeStruct((nw * LANES,), jnp.float32),
        compiler_params=pltpu.CompilerParams(
            use_tc_tiling_on_sc=True, needs_layout_passes=False),
        scratch_types=[
            pltpu.VMEM((b_per_w,), jnp.int32),
            pltpu.VMEM((FEAT_DIM, PITCH), jnp.float32),
            pltpu.VMEM((FEAT_DIM, b_per_w), jnp.float32),
            pltpu.VMEM((FEAT_DIM, TAIL_W), jnp.float32),
            pltpu.VMEM((LANES,), jnp.float32),
            pltpu.SemaphoreType.DMA,
        ],
    )
    def partials(featT_hbm, slab_hbm, ct_hbm, out_hbm,
                 labv, wbuf, fb, tailbuf, acc_v, wsem):
        wid = lax.axis_index("s") * nc + lax.axis_index("c")
        base = pl.multiple_of(wid * b_per_w, 128)
        obase = pl.multiple_of(wid * LANES, 8)

        pltpu.sync_copy(slab_hbm.at[pl.ds(base, b_per_w)], labv)
        pltpu.sync_copy(featT_hbm.at[:, pl.ds(base, b_per_w)], fb)
        pltpu.sync_copy(ct_hbm.at[:, pl.ds(TAIL0, TAIL_W)], tailbuf)

        zero = jnp.zeros((LANES,), jnp.float32)
        lanes_i = lax.iota(jnp.int32, LANES)
        maxp = b_per_w - 1

        def process(buf, wlo, hi, ptr, accs):
            a = list(accs)
            cnt = jnp.int32(0)
            for half in range(3):
                p = ptr + half * LANES + lanes_i
                cpos = jnp.minimum(p, maxp)
                lv = plsc.load_gather(labv, [cpos])
                sel = jnp.logical_and(p < b_per_w, lv < hi)
                sf = jnp.where(sel, 1.0, 0.0).astype(jnp.float32)
                off = jnp.clip(lv - wlo, 0, buf.shape[1] - 1)
                for f in range(FEAT_DIM):
                    fsplat = jnp.full((LANES,), f, jnp.int32)
                    cvec = plsc.load_gather(buf, [fsplat, off])
                    fvec = plsc.load_gather(fb, [fsplat, cpos])
                    d = fvec - cvec
                    a[f % 4] = a[f % 4] + sf * (d * d)
                cnt = cnt + jnp.sum(sel.astype(jnp.int32))
            return tuple(a), cnt

        def next_lab(nptr):
            cpos = jnp.minimum(nptr + lanes_i, maxp)
            return jnp.min(plsc.load_gather(labv, [cpos]))

        def main_cond(st):
            ptr, lab = st[0], st[1]
            return jnp.logical_and(ptr < b_per_w, lab < TAIL0)

        def main_body(st):
            ptr, lab = st[0], st[1]
            cw = jnp.minimum((lab >> 7) << 7, WSTART_MAX)
            cw = pl.multiple_of(cw, 128)
            cp = pltpu.make_async_copy(ct_hbm.at[:, pl.ds(cw, WIN)],
                                       wbuf.at[:, pl.ds(0, WIN)], wsem)
            cp.start()
            cp.wait()
            accs, cnt = process(wbuf, cw, cw + WIN, ptr, st[2:])
            nptr = ptr + cnt
            return (nptr, next_lab(nptr)) + accs

        def tail_cond(st):
            return st[0] < b_per_w

        def tail_body(st):
            ptr = st[0]
            accs, cnt = process(tailbuf, TAIL0, NCLASS, ptr, st[2:])
            nptr = ptr + cnt
            return (nptr, st[1]) + accs

        st0 = (jnp.int32(0), next_lab(jnp.int32(0)), zero, zero, zero, zero)
        st1 = lax.while_loop(main_cond, main_body, st0)
        st2 = lax.while_loop(tail_cond, tail_body, st1)

        a0, a1, a2, a3 = st2[2:]
        acc_v[...] = (a0 + a1) + (a2 + a3)
        pltpu.sync_copy(acc_v, out_hbm.at[pl.ds(obase, LANES)])

    return partials, nw


def kernel(feat, label, centers):
    partials, nw = _make_partials()
    iot = lax.iota(jnp.int32, BATCH)
    slab, order = lax.sort_key_val(label, iot)
    feat_s = jnp.take(feat, order, axis=0)
    parts = partials(feat_s.T, slab, centers.T)
    total = jnp.sum(parts)
    return LAMBDA_C / 2.0 / BATCH * jnp.sqrt(total)


# speedup vs baseline: 1.7826x; 1.0136x over previous
import functools

import jax
import jax.numpy as jnp
from jax import lax
from jax.experimental import pallas as pl
from jax.experimental.pallas import tpu as pltpu
from jax.experimental.pallas import tpu_sc as plsc

FEAT_DIM = 64
BATCH = 16384
NCLASS = 1000000
LAMBDA_C = 2.0
LANES = 16
WIN = 1152
PITCH = 1153
WSTART_MAX = ((NCLASS - WIN) // 128) * 128
TAIL0 = (NCLASS // 128) * 128
TAIL_W = NCLASS - TAIL0


def _make_partials():
    info = plsc.get_sparse_core_info()
    nc, ns = info.num_cores, info.num_subcores
    nw = nc * ns
    b_per_w = BATCH // nw

    mesh = plsc.VectorSubcoreMesh(core_axis_name="c", subcore_axis_name="s")

    @functools.partial(
        pl.kernel,
        mesh=mesh,
        out_type=jax.ShapeDtypeStruct((nw * LANES,), jnp.float32),
        compiler_params=pltpu.CompilerParams(
            use_tc_tiling_on_sc=True, needs_layout_passes=False),
        scratch_types=[
            pltpu.VMEM((b_per_w,), jnp.int32),
            pltpu.VMEM((FEAT_DIM, PITCH), jnp.float32),
            pltpu.VMEM((FEAT_DIM, b_per_w), jnp.float32),
            pltpu.VMEM((FEAT_DIM, TAIL_W), jnp.float32),
            pltpu.VMEM((LANES,), jnp.float32),
            pltpu.SemaphoreType.DMA,
        ],
    )
    def partials(featT_hbm, slab_hbm, ct_hbm, out_hbm,
                 labv, wbuf, fb, tailbuf, acc_v, wsem):
        wid = lax.axis_index("s") * nc + lax.axis_index("c")
        base = pl.multiple_of(wid * b_per_w, 128)
        obase = pl.multiple_of(wid * LANES, 8)

        pltpu.sync_copy(slab_hbm.at[pl.ds(base, b_per_w)], labv)
        pltpu.sync_copy(featT_hbm.at[:, pl.ds(base, b_per_w)], fb)
        pltpu.sync_copy(ct_hbm.at[:, pl.ds(TAIL0, TAIL_W)], tailbuf)

        zero = jnp.zeros((LANES,), jnp.float32)
        lanes_i = lax.iota(jnp.int32, LANES)
        maxp = b_per_w - 1

        def process(buf, wlo, hi, ptr, accs):
            a = list(accs)
            cnt = jnp.int32(0)
            for half in range(2):
                p = ptr + half * LANES + lanes_i
                cpos = jnp.minimum(p, maxp)
                lv = plsc.load_gather(labv, [cpos])
                sel = jnp.logical_and(p < b_per_w, lv < hi)
                sf = jnp.where(sel, 1.0, 0.0).astype(jnp.float32)
                off = jnp.clip(lv - wlo, 0, buf.shape[1] - 1)
                for f in range(FEAT_DIM):
                    fsplat = jnp.full((LANES,), f, jnp.int32)
                    cvec = plsc.load_gather(buf, [fsplat, off])
                    fvec = plsc.load_gather(fb, [fsplat, cpos])
                    d = fvec - cvec
                    a[f % 4] = a[f % 4] + sf * (d * d)
                cnt = cnt + jnp.sum(sel.astype(jnp.int32))
            return tuple(a), cnt

        def next_lab(nptr):
            cpos = jnp.minimum(nptr + lanes_i, maxp)
            return jnp.min(plsc.load_gather(labv, [cpos]))

        def main_cond(st):
            ptr, lab = st[0], st[1]
            return jnp.logical_and(ptr < b_per_w, lab < TAIL0)

        def main_body(st):
            ptr, lab = st[0], st[1]
            cw = jnp.minimum((lab >> 7) << 7, WSTART_MAX)
            cw = pl.multiple_of(cw, 128)
            cp = pltpu.make_async_copy(ct_hbm.at[:, pl.ds(cw, WIN)],
                                       wbuf.at[:, pl.ds(0, WIN)], wsem)
            cp.start()
            cp.wait()
            accs, cnt = process(wbuf, cw, cw + WIN, ptr, st[2:])
            nptr = ptr + cnt
            return (nptr, next_lab(nptr)) + accs

        def tail_cond(st):
            return st[0] < b_per_w

        def tail_body(st):
            ptr = st[0]
            accs, cnt = process(tailbuf, TAIL0, NCLASS, ptr, st[2:])
            nptr = ptr + cnt
            return (nptr, st[1]) + accs

        st0 = (jnp.int32(0), next_lab(jnp.int32(0)), zero, zero, zero, zero)
        st1 = lax.while_loop(main_cond, main_body, st0)
        st2 = lax.while_loop(tail_cond, tail_body, st1)

        a0, a1, a2, a3 = st2[2:]
        acc_v[...] = (a0 + a1) + (a2 + a3)
        pltpu.sync_copy(acc_v, out_hbm.at[pl.ds(obase, LANES)])

    return partials, nw


def kernel(feat, label, centers):
    partials, nw = _make_partials()
    iot = lax.iota(jnp.int32, BATCH)
    slab, order = lax.sort_key_val(label, iot)
    feat_s = jnp.take(feat, order, axis=0)
    parts = partials(feat_s.T, slab, centers.T)
    total = jnp.sum(parts)
    return LAMBDA_C / 2.0 / BATCH * jnp.sqrt(total)
